# baseline (speedup 1.0000x reference)
import jax
import jax.numpy as jnp
from jax import lax
from jax.experimental import pallas as pl
from jax.experimental.pallas import tpu as pltpu
from jax.experimental.pallas import tpu_sc as plsc

B = 4096
L = 50
H = 100
OUT = 128

NC = 2
NS = 16
NW = NC * NS
BSC = 2688
BTC = B - BSC
BPW = BSC // NW

_OFFS = (0, 16, 32, 48, 64, 80, 84)
_SLOT = [(k // 16, k % 16) if k < 48 else (3, k - 34) for k in range(L)]
ROWS_BUF = 8 * L + 8


def _sc_pool_body(idx_hbm, table_hbm, out_hbm, idx_v, rows_v, acc_v, sems):
    wid = lax.axis_index("s") * NC + lax.axis_index("c")

    pltpu.sync_copy(idx_hbm.at[pl.ds(wid * (BPW * L), BPW * L)], idx_v)

    def enqueue(cc):
        buf = lax.bitwise_and(cc, 1)
        iv = [idx_v[pl.ds(cc * L + o, 16)] for o in (0, 16, 32, 34)]
        for k in range(L):
            j, lane = _SLOT[k]
            i = iv[j][lane]
            off = pl.multiple_of(lax.bitwise_and(i, -8), 8)
            d = 8 * k + 7 - (i - off)
            pltpu.async_copy(
                table_hbm.at[pl.ds(off, 8)],
                rows_v.at[buf, pl.ds(d, 8)],
                sems.at[buf],
            )

    enqueue(0)

    def chunk_body(c, _):
        @pl.when(c + 1 < BPW)
        def _pre():
            enqueue(c + 1)

        buf = lax.bitwise_and(c, 1)
        pltpu.make_async_copy(
            table_hbm.at[pl.ds(0, 8 * L)],
            rows_v.at[buf, pl.ds(0, 8 * L)],
            sems.at[buf],
        ).wait()

        accs = [rows_v[buf, 7, pl.ds(_OFFS[s], 16)] for s in range(len(_OFFS))]
        for w in range(1, L):
            for s in range(len(_OFFS)):
                accs[s] = accs[s] + rows_v[buf, 8 * w + 7, pl.ds(_OFFS[s], 16)]
        for s in range(len(_OFFS)):
            acc_v[pl.ds(c * H + _OFFS[s], 16)] = accs[s]
        return 0

    lax.fori_loop(0, BPW, chunk_body, 0)

    pltpu.sync_copy(acc_v, out_hbm.at[pl.ds(wid * (BPW * H), BPW * H)])


def _sc_pool(idx_flat, table):
    mesh = plsc.VectorSubcoreMesh(core_axis_name="c", subcore_axis_name="s")
    return pl.kernel(
        _sc_pool_body,
        out_type=jax.ShapeDtypeStruct((BSC * H,), jnp.float32),
        mesh=mesh,
        scratch_types=[
            pltpu.VMEM((BPW * L,), jnp.int32),
            pltpu.VMEM((2, ROWS_BUF, H), jnp.float32),
            pltpu.VMEM((BPW * H,), jnp.float32),
            pltpu.SemaphoreType.DMA((2,)),
        ],
    )(idx_flat, table)


NBUF = 16
RPP = 8


def _tc_pool_body(idx_ref, table_ref, o_ref, buf, sems):
    iota8 = lax.broadcasted_iota(jnp.int32, (8, 1), 0)

    def fetch(t):
        r, k = divmod(t, L)
        i = idx_ref[r, k]
        off = pl.multiple_of(jnp.bitwise_and(i, -8), 8)
        slot = t % NBUF
        pltpu.make_async_copy(
            table_ref.at[pl.ds(off, 8)], buf.at[slot], sems.at[slot]
        ).start()

    for t in range(NBUF):
        fetch(t)
    for r in range(RPP):
        acc = jnp.zeros((8, H), jnp.float32)
        for k in range(L):
            t = r * L + k
            slot = t % NBUF
            pltpu.make_async_copy(
                table_ref.at[pl.ds(0, 8)], buf.at[slot], sems.at[slot]
            ).wait()
            i = idx_ref[r, k]
            m = i - jnp.bitwise_and(i, -8)
            blk = buf[slot]
            acc = acc + jnp.where(iota8 == m, blk, 0.0)
            if t + NBUF < RPP * L:
                fetch(t + NBUF)
        o_ref[pl.ds(r, 1), :] = jnp.sum(acc, axis=0, keepdims=True)


def _tc_pool(idx_tc, table):
    return pl.pallas_call(
        _tc_pool_body,
        grid=(BTC // RPP,),
        in_specs=[
            pl.BlockSpec((RPP, L), lambda i: (i, 0), memory_space=pltpu.SMEM),
            pl.BlockSpec(memory_space=pltpu.MemorySpace.HBM),
        ],
        out_specs=pl.BlockSpec((RPP, H), lambda i: (i, 0)),
        out_shape=jax.ShapeDtypeStruct((BTC, H), jnp.float32),
        scratch_shapes=[
            pltpu.VMEM((NBUF, 8, H), jnp.float32),
            pltpu.SemaphoreType.DMA((NBUF,)),
        ],
    )(idx_tc, table)


def _tc_matmul_body(x_ref, w_ref, b_ref, o_ref):
    x = x_ref[...] * (1.0 / L)
    o_ref[...] = (
        lax.dot_general(
            x, w_ref[...], (((1,), (1,)), ((), ())),
            preferred_element_type=jnp.float32,
        )
        + b_ref[...]
    )


def _tc_matmul(pooled, W, b2d):
    return pl.pallas_call(
        _tc_matmul_body,
        out_shape=jax.ShapeDtypeStruct((B, OUT), jnp.float32),
    )(pooled, W, b2d)


def kernel(keyword_tensor_list, word_embed, W, b):
    idx = keyword_tensor_list.astype(jnp.int32)
    p_sc = _sc_pool(idx[:BSC].reshape(BSC * L), word_embed).reshape(BSC, H)
    p_tc = _tc_pool(idx[BSC:], word_embed)
    pooled = jnp.concatenate([p_sc, p_tc], axis=0)
    return _tc_matmul(pooled, W, b.reshape(1, OUT))


# speedup vs baseline: 3.5030x; 3.5030x over previous
import jax
import jax.numpy as jnp
from jax import lax
from jax.experimental import pallas as pl
from jax.experimental.pallas import tpu as pltpu
from jax.experimental.pallas import tpu_sc as plsc

B = 4096
L = 50
H = 100
OUT = 128

NC = 2
NS = 16
NW = NC * NS
BSC = 3840
BTC = B - BSC
BPW = BSC // NW

_OFFS = (0, 16, 32, 48, 64, 80, 84)
_SLOT = [(k // 16, k % 16) if k < 48 else (3, k - 34) for k in range(L)]
ROWS_BUF = 8 * L + 8


def _sc_pool_body(idx_hbm, table_hbm, out_hbm, idx_v, rows_v, acc_v, sems):
    wid = lax.axis_index("s") * NC + lax.axis_index("c")

    pltpu.sync_copy(idx_hbm.at[pl.ds(wid * (BPW * L), BPW * L)], idx_v)

    def enqueue(cc):
        buf = lax.bitwise_and(cc, 1)
        iv = [idx_v[pl.ds(cc * L + o, 16)] for o in (0, 16, 32, 34)]
        for k in range(L):
            j, lane = _SLOT[k]
            i = iv[j][lane]
            off = pl.multiple_of(lax.bitwise_and(i, -8), 8)
            d = 8 * k + 7 - (i - off)
            pltpu.async_copy(
                table_hbm.at[pl.ds(off, 8)],
                rows_v.at[buf, pl.ds(d, 8)],
                sems.at[buf],
            )

    enqueue(0)

    def chunk_body(c, _):
        @pl.when(c + 1 < BPW)
        def _pre():
            enqueue(c + 1)

        buf = lax.bitwise_and(c, 1)
        pltpu.make_async_copy(
            table_hbm.at[pl.ds(0, 8 * L)],
            rows_v.at[buf, pl.ds(0, 8 * L)],
            sems.at[buf],
        ).wait()

        accs = [rows_v[buf, 7, pl.ds(_OFFS[s], 16)] for s in range(len(_OFFS))]
        for w in range(1, L):
            for s in range(len(_OFFS)):
                accs[s] = accs[s] + rows_v[buf, 8 * w + 7, pl.ds(_OFFS[s], 16)]
        for s in range(len(_OFFS)):
            acc_v[pl.ds(c * H + _OFFS[s], 16)] = accs[s]
        return 0

    lax.fori_loop(0, BPW, chunk_body, 0)

    pltpu.sync_copy(acc_v, out_hbm.at[pl.ds(wid * (BPW * H), BPW * H)])


def _sc_pool(idx_flat, table):
    mesh = plsc.VectorSubcoreMesh(core_axis_name="c", subcore_axis_name="s")
    return pl.kernel(
        _sc_pool_body,
        out_type=jax.ShapeDtypeStruct((BSC * H,), jnp.float32),
        mesh=mesh,
        scratch_types=[
            pltpu.VMEM((BPW * L,), jnp.int32),
            pltpu.VMEM((2, ROWS_BUF, H), jnp.float32),
            pltpu.VMEM((BPW * H,), jnp.float32),
            pltpu.SemaphoreType.DMA((2,)),
        ],
    )(idx_flat, table)


NBUF = 16
RPP = 8


def _tc_pool_body(idx_ref, table_ref, o_ref, buf, sems):
    iota8 = lax.broadcasted_iota(jnp.int32, (8, 1), 0)

    def fetch(t):
        r, k = divmod(t, L)
        i = idx_ref[r, k]
        off = pl.multiple_of(jnp.bitwise_and(i, -8), 8)
        slot = t % NBUF
        pltpu.make_async_copy(
            table_ref.at[pl.ds(off, 8)], buf.at[slot], sems.at[slot]
        ).start()

    for t in range(NBUF):
        fetch(t)
    for r in range(RPP):
        acc = jnp.zeros((8, H), jnp.float32)
        for k in range(L):
            t = r * L + k
            slot = t % NBUF
            pltpu.make_async_copy(
                table_ref.at[pl.ds(0, 8)], buf.at[slot], sems.at[slot]
            ).wait()
            i = idx_ref[r, k]
            m = i - jnp.bitwise_and(i, -8)
            blk = buf[slot]
            acc = acc + jnp.where(iota8 == m, blk, 0.0)
            if t + NBUF < RPP * L:
                fetch(t + NBUF)
        o_ref[pl.ds(r, 1), :] = jnp.sum(acc, axis=0, keepdims=True)


def _tc_pool(idx_tc, table):
    return pl.pallas_call(
        _tc_pool_body,
        grid=(BTC // RPP,),
        in_specs=[
            pl.BlockSpec((RPP, L), lambda i: (i, 0), memory_space=pltpu.SMEM),
            pl.BlockSpec(memory_space=pltpu.MemorySpace.HBM),
        ],
        out_specs=pl.BlockSpec((RPP, H), lambda i: (i, 0)),
        out_shape=jax.ShapeDtypeStruct((BTC, H), jnp.float32),
        scratch_shapes=[
            pltpu.VMEM((NBUF, 8, H), jnp.float32),
            pltpu.SemaphoreType.DMA((NBUF,)),
        ],
    )(idx_tc, table)


def _tc_matmul_body(x_ref, w_ref, b_ref, o_ref):
    x = x_ref[...] * (1.0 / L)
    o_ref[...] = (
        lax.dot_general(
            x, w_ref[...], (((1,), (1,)), ((), ())),
            preferred_element_type=jnp.float32,
        )
        + b_ref[...]
    )


def _tc_matmul(pooled, W, b2d):
    return pl.pallas_call(
        _tc_matmul_body,
        out_shape=jax.ShapeDtypeStruct((B, OUT), jnp.float32),
    )(pooled, W, b2d)


def kernel(keyword_tensor_list, word_embed, W, b):
    idx = keyword_tensor_list.astype(jnp.int32)
    p_sc = _sc_pool(idx[:BSC].reshape(BSC * L), word_embed).reshape(BSC, H)
    p_tc = _tc_pool(idx[BSC:], word_embed)
    pooled = jnp.concatenate([p_sc, p_tc], axis=0)
    return _tc_matmul(pooled, W, b.reshape(1, OUT))


# speedup vs baseline: 5.0104x; 1.4303x over previous
import jax
import jax.numpy as jnp
from jax import lax
from jax.experimental import pallas as pl
from jax.experimental.pallas import tpu as pltpu
from jax.experimental.pallas import tpu_sc as plsc

B = 4096
L = 50
H = 100
OUT = 128

NC = 2
NS = 16
NW = NC * NS
BPW = B // NW

_OFFS = (0, 16, 32, 48, 64, 80, 84)
_SLOT = [(k // 16, k % 16) if k < 48 else (3, k - 34) for k in range(L)]
ROWS_BUF = 8 * L + 8


def _sc_pool_body(idx_hbm, table_hbm, out_hbm, idx_v, rows_v, acc_v, sems):
    wid = lax.axis_index("s") * NC + lax.axis_index("c")

    pltpu.sync_copy(idx_hbm.at[pl.ds(wid * (BPW * L), BPW * L)], idx_v)

    def enqueue(cc):
        buf = lax.bitwise_and(cc, 1)
        iv = [idx_v[pl.ds(cc * L + o, 16)] for o in (0, 16, 32, 34)]
        for k in range(L):
            j, lane = _SLOT[k]
            i = iv[j][lane]
            off = pl.multiple_of(lax.bitwise_and(i, -8), 8)
            d = 8 * k + 7 - (i - off)
            pltpu.async_copy(
                table_hbm.at[pl.ds(off, 8)],
                rows_v.at[buf, pl.ds(d, 8)],
                sems.at[buf],
            )

    enqueue(0)

    def chunk_body(c, _):
        @pl.when(c + 1 < BPW)
        def _pre():
            enqueue(c + 1)

        buf = lax.bitwise_and(c, 1)
        pltpu.make_async_copy(
            table_hbm.at[pl.ds(0, 8 * L)],
            rows_v.at[buf, pl.ds(0, 8 * L)],
            sems.at[buf],
        ).wait()

        accs = [rows_v[buf, 7, pl.ds(_OFFS[s], 16)] for s in range(len(_OFFS))]
        for w in range(1, L):
            for s in range(len(_OFFS)):
                accs[s] = accs[s] + rows_v[buf, 8 * w + 7, pl.ds(_OFFS[s], 16)]
        for s in range(len(_OFFS)):
            acc_v[pl.ds(c * H + _OFFS[s], 16)] = accs[s]
        return 0

    lax.fori_loop(0, BPW, chunk_body, 0)

    pltpu.sync_copy(acc_v, out_hbm.at[pl.ds(wid * (BPW * H), BPW * H)])


def _sc_pool(idx_flat, table):
    mesh = plsc.VectorSubcoreMesh(core_axis_name="c", subcore_axis_name="s")
    return pl.kernel(
        _sc_pool_body,
        out_type=jax.ShapeDtypeStruct((B * H,), jnp.float32),
        mesh=mesh,
        scratch_types=[
            pltpu.VMEM((BPW * L,), jnp.int32),
            pltpu.VMEM((2, ROWS_BUF, H), jnp.float32),
            pltpu.VMEM((BPW * H,), jnp.float32),
            pltpu.SemaphoreType.DMA((2,)),
        ],
    )(idx_flat, table)


def _tc_matmul_body(x_ref, w_ref, b_ref, o_ref):
    x = x_ref[...] * (1.0 / L)
    o_ref[...] = (
        lax.dot_general(
            x, w_ref[...], (((1,), (1,)), ((), ())),
            preferred_element_type=jnp.float32,
        )
        + b_ref[...]
    )


def _tc_matmul(pooled, W, b2d):
    return pl.pallas_call(
        _tc_matmul_body,
        out_shape=jax.ShapeDtypeStruct((B, OUT), jnp.float32),
    )(pooled, W, b2d)


def kernel(keyword_tensor_list, word_embed, W, b):
    idx_flat = keyword_tensor_list.astype(jnp.int32).reshape(B * L)
    pooled = _sc_pool(idx_flat, word_embed).reshape(B, H)
    return _tc_matmul(pooled, W, b.reshape(1, OUT))
